# baseline (speedup 1.0000x reference)
import functools

import jax
import jax.numpy as jnp
from jax import lax
from jax.experimental import pallas as pl
from jax.experimental.pallas import tpu as pltpu
from jax.experimental.pallas import tpu_sc as plsc

_BM = 2048
_ROWS_PER_WORKER = 8


def _sc_gather_body(table_ref, ids_ref, out_ref, idx_v, rows_v, sem):
    nc = 2
    wid = lax.axis_index("s") * nc + lax.axis_index("c")
    nw_used = out_ref.shape[0] // _ROWS_PER_WORKER

    @pl.when(wid < nw_used)
    def _():
        base = wid * _ROWS_PER_WORKER
        pltpu.sync_copy(ids_ref.at[pl.ds(base, _ROWS_PER_WORKER)], idx_v)
        pltpu.async_copy(table_ref.at[idx_v], rows_v, sem).wait()
        pltpu.sync_copy(rows_v, out_ref.at[pl.ds(base, _ROWS_PER_WORKER)])


def _sc_gather(degree_counts_by_id, embodiment_ids):
    batch = embodiment_ids.shape[0]
    seq_len = degree_counts_by_id.shape[1]
    mesh = plsc.VectorSubcoreMesh(core_axis_name="c", subcore_axis_name="s")
    return pl.kernel(
        _sc_gather_body,
        out_type=jax.ShapeDtypeStruct((batch, seq_len), jnp.int32),
        mesh=mesh,
        scratch_types=[
            pltpu.VMEM((_ROWS_PER_WORKER,), jnp.int32),
            pltpu.VMEM((_ROWS_PER_WORKER, seq_len), jnp.int32),
            pltpu.SemaphoreType.DMA,
        ],
    )(degree_counts_by_id, embodiment_ids)


def _tc_body(dc_ref, emb_ref, tok_ref, out_ref):
    bm = tok_ref.shape[0]
    bs = dc_ref.shape[1]
    nb = dc_ref.shape[2]
    n_rows = emb_ref.shape[0]

    dc_sb = dc_ref[0].astype(jnp.float32)

    rs_io = lax.broadcasted_iota(jnp.int32, (bm, bs), 0)
    s_io = lax.broadcasted_iota(jnp.int32, (bm, bs), 1)
    rep_s = (rs_io // nb == s_io).astype(jnp.float32)
    tmp = jnp.dot(rep_s, dc_sb, preferred_element_type=jnp.float32)

    r_io = lax.broadcasted_iota(jnp.int32, (bm, nb), 0)
    b_io = lax.broadcasted_iota(jnp.int32, (bm, nb), 1)
    mask_b = (r_io % nb == b_io).astype(jnp.float32)
    idx = jnp.sum(tmp * mask_b, axis=1, keepdims=True)

    k_io = lax.broadcasted_iota(jnp.int32, (bm, n_rows), 1).astype(jnp.float32)
    oh = (idx == k_io).astype(jnp.bfloat16)
    pe = jnp.dot(oh, emb_ref[...], preferred_element_type=jnp.float32)
    out_ref[...] = tok_ref[...] + pe


def kernel(tokens, embodiment_ids, degree_counts_by_id, embedding):
    seq_len, batch, d_model = tokens.shape
    n_rows = embedding.shape[0]
    m = seq_len * batch
    bs = _BM // batch

    degree_counts = _sc_gather(degree_counts_by_id, embodiment_ids)
    dc3 = degree_counts.T.reshape(m // _BM, bs, batch)

    tok2 = tokens.reshape(m, d_model)
    emb_bf = embedding.astype(jnp.bfloat16)

    out2 = pl.pallas_call(
        _tc_body,
        grid=(m // _BM,),
        in_specs=[
            pl.BlockSpec((1, bs, batch), lambda i: (i, 0, 0)),
            pl.BlockSpec((n_rows, d_model), lambda i: (0, 0)),
            pl.BlockSpec((_BM, d_model), lambda i: (i, 0)),
        ],
        out_specs=pl.BlockSpec((_BM, d_model), lambda i: (i, 0)),
        out_shape=jax.ShapeDtypeStruct((m, d_model), jnp.float32),
    )(dc3, emb_bf, tok2)
    return out2.reshape(seq_len, batch, d_model)
